# baseline (speedup 1.0000x reference)
import jax
import jax.numpy as jnp
from jax import lax
from jax.experimental import pallas as pl
from jax.experimental.pallas import tpu as pltpu
from jax.experimental.pallas import tpu_sc as plsc

VOCAB = 100000
EMBED = 64
BATCH = 16384
LANES = 16
NC = 2
NS = 16
NW = NC * NS
BPW = BATCH // NW
CHUNK = 128
NCHUNK = BPW // CHUNK
GROUPS = BPW // LANES


def _glove_body(wu_hbm, wv_hbm, in_embed_hbm, in_bias_hbm, out_embed_hbm,
                out_bias_hbm, out_hbm, idx_u, idx_v, u_rows, v_rows,
                u_bias, v_bias, out_buf, sem):
    wid = lax.axis_index("s") * NC + lax.axis_index("c")
    base_row = wid * NCHUNK

    pltpu.sync_copy(wu_hbm.at[pl.ds(base_row, NCHUNK)], idx_u)
    pltpu.sync_copy(wv_hbm.at[pl.ds(base_row, NCHUNK)], idx_v)

    copies = []
    for j in range(NCHUNK):
        sl = pl.ds(j * CHUNK, CHUNK)
        copies.append(pltpu.make_async_copy(
            in_embed_hbm.at[idx_u.at[j]], u_rows.at[sl], sem))
        copies.append(pltpu.make_async_copy(
            out_embed_hbm.at[idx_v.at[j]], v_rows.at[sl], sem))
        copies.append(pltpu.make_async_copy(
            in_bias_hbm.at[idx_u.at[j]], u_bias.at[sl], sem))
        copies.append(pltpu.make_async_copy(
            out_bias_hbm.at[idx_v.at[j]], v_bias.at[sl], sem))
    for c in copies:
        c.start()
    for c in copies:
        c.wait()

    lane = lax.iota(jnp.int32, LANES)

    def group(g, carry):
        rbase = g * LANES
        acc = jnp.zeros((LANES,), jnp.float32)
        for r in range(LANES):
            row = rbase + r
            s = (u_rows[row, pl.ds(0, 16)] * v_rows[row, pl.ds(0, 16)]
                 + u_rows[row, pl.ds(16, 16)] * v_rows[row, pl.ds(16, 16)]
                 + u_rows[row, pl.ds(32, 16)] * v_rows[row, pl.ds(32, 16)]
                 + u_rows[row, pl.ds(48, 16)] * v_rows[row, pl.ds(48, 16)])
            total = jnp.sum(s, axis=0)
            acc = jnp.where(lane == r, total, acc)
        sl16 = pl.ds(rbase, LANES)
        out_buf[sl16] = acc + u_bias[sl16] + v_bias[sl16]
        return carry

    lax.fori_loop(0, GROUPS, group, 0)
    pltpu.sync_copy(out_buf, out_hbm.at[pl.ds(wid * BPW, BPW)])


def _glove_sc(word_u2d, word_v2d, in_embed, in_bias1d, out_embed, out_bias1d):
    mesh = plsc.VectorSubcoreMesh(core_axis_name="c", subcore_axis_name="s")
    f = pl.kernel(
        _glove_body,
        out_type=jax.ShapeDtypeStruct((BATCH,), jnp.float32),
        mesh=mesh,
        scratch_types=[
            pltpu.VMEM((NCHUNK, CHUNK), jnp.int32),
            pltpu.VMEM((NCHUNK, CHUNK), jnp.int32),
            pltpu.VMEM((BPW, EMBED), jnp.float32),
            pltpu.VMEM((BPW, EMBED), jnp.float32),
            pltpu.VMEM((BPW,), jnp.float32),
            pltpu.VMEM((BPW,), jnp.float32),
            pltpu.VMEM((BPW,), jnp.float32),
            pltpu.SemaphoreType.DMA,
        ],
        compiler_params=pltpu.CompilerParams(
            needs_layout_passes=False, use_tc_tiling_on_sc=False),
    )
    return f(word_u2d, word_v2d, in_embed, in_bias1d, out_embed, out_bias1d)


def kernel(word_u, word_v, in_embed, in_bias, out_embed, out_bias):
    wu = word_u.astype(jnp.int32).reshape(NW * NCHUNK, CHUNK)
    wv = word_v.astype(jnp.int32).reshape(NW * NCHUNK, CHUNK)
    return _glove_sc(wu, wv, in_embed, in_bias.reshape(VOCAB),
                     out_embed, out_bias.reshape(VOCAB))
